# baseline (speedup 1.0000x reference)
import functools

import jax
import jax.numpy as jnp
from jax import lax
from jax.experimental import pallas as pl
from jax.experimental.pallas import tpu as pltpu
from jax.experimental.pallas import tpu_sc as plsc

_N = 10000
_E = 320000
_G = 64
_NPAD = 10240
_NW = 32
_CH = 128
_RPW = 80
_ROWS = _NW * _RPW
_EPAD = _ROWS * _CH
_NBUF = 8


def _sc_agg(d, with_deg):
    mesh = plsc.VectorSubcoreMesh(core_axis_name="c", subcore_axis_name="s")
    nacc = (2 * _NPAD) if with_deg else _NPAD
    rpt = _NPAD // 16

    scratch = [
        pltpu.VMEM((_RPW, _CH), jnp.int32),
        pltpu.VMEM((_RPW, _CH), jnp.int32),
        pltpu.VMEM((_NBUF, _CH, d), jnp.float32),
        pltpu.VMEM_SHARED((_NPAD, d), jnp.float32),
    ]
    scratch += [pltpu.SemaphoreType.DMA] * (2 * _NBUF)
    if with_deg:
        scratch += [pltpu.VMEM((_CH, d), jnp.float32),
                    pltpu.VMEM_SHARED((_NPAD, d), jnp.float32)]
        scratch += [pltpu.SemaphoreType.DMA] * _NBUF

    def body(*refs):
        if with_deg:
            (y_hbm, srcr_hbm, dstr_hbm, zeros_hbm, ones_hbm, out_hbm,
             sidx, didx, rows, acc, *sems) = refs
            gsems = sems[:_NBUF]
            ssems = sems[_NBUF:2 * _NBUF]
            ones, dacc, *s2sems = sems[2 * _NBUF:]
        else:
            (y_hbm, srcr_hbm, dstr_hbm, zeros_hbm, out_hbm,
             sidx, didx, rows, acc, *sems) = refs
            gsems = sems[:_NBUF]
            ssems = sems[_NBUF:2 * _NBUF]

        c = lax.axis_index("c")
        s = lax.axis_index("s")
        wid = s * 2 + c
        pltpu.sync_copy(zeros_hbm.at[pl.ds(s * rpt, rpt)], acc.at[pl.ds(s * rpt, rpt)])
        if with_deg:
            pltpu.sync_copy(zeros_hbm.at[pl.ds(s * rpt, rpt)], dacc.at[pl.ds(s * rpt, rpt)])
        base = wid * _RPW
        pltpu.sync_copy(srcr_hbm.at[pl.ds(base, _RPW)], sidx)
        pltpu.sync_copy(dstr_hbm.at[pl.ds(base, _RPW)], didx)
        if with_deg:
            pltpu.sync_copy(ones_hbm, ones)
        plsc.subcore_barrier()

        for b in range(_NBUF):
            pltpu.async_copy(y_hbm.at[sidx.at[b]], rows.at[b], gsems[b])

        def step(i, carry):
            for b in range(_NBUF):
                r = i * _NBUF + b
                pltpu.make_async_copy(y_hbm.at[sidx.at[r]], rows.at[b], gsems[b]).wait()
                sd = pltpu.async_copy(rows.at[b], acc.at[didx.at[r]], ssems[b], add=True)
                if with_deg:
                    sd2 = pltpu.async_copy(ones, dacc.at[didx.at[r]], s2sems[b], add=True)
                nxt = r + _NBUF

                @pl.when(nxt < _RPW)
                def _refill():
                    sd.wait()
                    if with_deg:
                        sd2.wait()
                    pltpu.async_copy(y_hbm.at[sidx.at[nxt]], rows.at[b], gsems[b])

            return carry

        lax.fori_loop(0, _RPW // _NBUF, step, 0)
        for b in range(_NBUF):
            r = _RPW - _NBUF + b
            pltpu.make_async_copy(rows.at[b], acc.at[didx.at[r]], ssems[b]).wait()
            if with_deg:
                pltpu.make_async_copy(ones, dacc.at[didx.at[r]], s2sems[b]).wait()
        plsc.subcore_barrier()
        pltpu.sync_copy(acc.at[pl.ds(s * rpt, rpt)],
                        out_hbm.at[c, pl.ds(s * rpt, rpt)])
        if with_deg:
            pltpu.sync_copy(dacc.at[pl.ds(s * rpt, rpt)],
                            out_hbm.at[c, pl.ds(_NPAD + s * rpt, rpt)])

    return functools.partial(
        pl.kernel,
        out_type=jax.ShapeDtypeStruct((2, nacc, d), jnp.float32),
        mesh=mesh,
        scratch_types=scratch,
        compiler_params=pltpu.CompilerParams(use_tc_tiling_on_sc=False),
    )(body)


def _mm(a, b):
    return lax.dot_general(a, b, (((1,), (0,)), ((), ())),
                           preferred_element_type=jnp.float32)


def _dot_t(a, b):
    return lax.dot_general(a, b, (((1,), (1,)), ((), ())),
                           preferred_element_type=jnp.float32)


_NP8 = _N // 8
_PP8 = _NPAD // 8
_NP2 = _N // 2
_PP2 = _NPAD // 2


def _tk1(x2, w1l_pk, w1r_pk):
    def body(x_ref, wl_ref, wr_ref, y_ref, z_ref):
        xv = x_ref[...]
        y_ref[...] = _mm(xv, wl_ref[...])
        z_ref[...] = _mm(xv, wr_ref[...])

    return pl.pallas_call(
        body,
        out_shape=(jax.ShapeDtypeStruct((_NP8, 128), jnp.float32),
                   jax.ShapeDtypeStruct((_NP8, 128), jnp.float32)),
    )(x2, w1l_pk, w1r_pk)


def _fold_bn(pre, tfold, n_nodes, g_t, be_t):
    s = jnp.sum(pre, axis=0, keepdims=True)
    sq = jnp.sum(pre * pre, axis=0, keepdims=True)
    mu = _mm(s, tfold) * (1.0 / n_nodes)
    ex2 = _mm(sq, tfold) * (1.0 / n_nodes)
    var = ex2 - mu * mu
    h = (pre - mu) * lax.rsqrt(var + 1e-5) * g_t + be_t
    return jnp.maximum(h, 0.0)


def _tk2(p, z1, b1_t, g1_t, be1_t, tf16):
    def body(p_ref, z_ref, b_ref, g_ref, be_ref, tf_ref, h_ref, dinv_ref):
        sm = p_ref[0] + p_ref[1]
        agg = sm[:_NP8, :]
        deg = sm[_PP8:_PP8 + _NP8, :]
        dinv = 1.0 / jnp.maximum(deg, 1.0)
        pre = agg * dinv + b_ref[...] + z_ref[...]
        h_ref[...] = _fold_bn(pre, tf_ref[...], _N, g_ref[...], be_ref[...])
        dinv_ref[...] = dinv

    return pl.pallas_call(
        body,
        out_shape=(jax.ShapeDtypeStruct((_NP8, 128), jnp.float32),
                   jax.ShapeDtypeStruct((_NP8, 128), jnp.float32)),
    )(p, z1, b1_t, g1_t, be1_t, tf16)


def _tk3(p, h1, w2l_pk, b2_t, w2r_pk, g2_t, be2_t, dinv, tf64):
    def body(p_ref, h1_ref, wl_ref, b_ref, wr_ref, g_ref, be_ref, dinv_ref,
             tf_ref, h2_ref):
        agg = (p_ref[0] + p_ref[1])[:_NP8, :]
        mean2 = agg * dinv_ref[...]
        pre = _mm(mean2, wl_ref[...]) + b_ref[...] + _mm(h1_ref[...], wr_ref[...])
        h2_ref[...] = _fold_bn(pre, tf_ref[...], _N, g_ref[...], be_ref[...])

    return pl.pallas_call(
        body,
        out_shape=jax.ShapeDtypeStruct((_NP8, 512), jnp.float32),
    )(p, h1, w2l_pk, b2_t, w2r_pk, g2_t, be2_t, dinv, tf64)


def _tk4r(h2pair, w3r_pk, b3_t):
    def body(h2_ref, wr_ref, b_ref, r_ref):
        r_ref[...] = _mm(h2_ref[...], wr_ref[...]) + b_ref[...]

    return pl.pallas_call(
        body,
        out_shape=jax.ShapeDtypeStruct((_NP2, 1024), jnp.float32),
    )(h2pair, w3r_pk, b3_t)


def _tk4a(p, root, w3l_pk, dd):
    def body(p_ref, root_ref, wl_ref, dd_ref, pre_ref, mu_ref, rv_ref):
        agg = (p_ref[0] + p_ref[1])[:_NP2, :]
        mm = _mm(agg, wl_ref[...])
        dd = dd_ref[...]
        mean3 = jnp.concatenate(
            [mm[:, :512] * dd[:, 0:1], mm[:, 512:] * dd[:, 16:17]], axis=1)
        pre = mean3 + root_ref[...]
        pre_ref[...] = pre
        s = jnp.sum(pre, axis=0, keepdims=True)
        sq = jnp.sum(pre * pre, axis=0, keepdims=True)
        sf = s[:, :512] + s[:, 512:]
        sqf = sq[:, :512] + sq[:, 512:]
        mu = jnp.concatenate([sf, sf], axis=1) * (1.0 / _N)
        ex2 = jnp.concatenate([sqf, sqf], axis=1) * (1.0 / _N)
        mu_ref[...] = mu
        rv_ref[...] = lax.rsqrt(ex2 - mu * mu + 1e-5)

    return pl.pallas_call(
        body,
        out_shape=(jax.ShapeDtypeStruct((_NP2, 1024), jnp.float32),
                   jax.ShapeDtypeStruct((1, 1024), jnp.float32),
                   jax.ShapeDtypeStruct((1, 1024), jnp.float32)),
    )(p, root, w3l_pk, dd)


_BLK = 200
_NBLK = _NP2 // _BLK


def _tk4b(pre, mu, rv, g3_t, be3_t, bb, wf1, bf1, wf2, bf2):
    def body(mu_ref, rv_ref, g_ref, be_ref, wf1_ref, bf1_ref, wf2_ref, bf2_ref,
             pre_ref, bb_ref, out_ref, pooled_ref):
        i = pl.program_id(0)

        @pl.when(i == 0)
        def _init():
            pooled_ref[...] = jnp.full((_G, 512), -jnp.inf, jnp.float32)

        h = pre_ref[...]
        h = (h - mu_ref[...]) * rv_ref[...] * g_ref[...] + be_ref[...]
        h = jnp.maximum(h, 0.0)
        hl = h[:, :512]
        hr = h[:, 512:]
        bbv = bb_ref[...]
        bb0 = bbv[:, 0:1]
        bb1 = bbv[:, 1:2]
        bmin = jnp.min(bb0)
        bmax = jnp.max(bb1)

        def seg_body(g, carry):
            redl = jnp.max(jnp.where(bb0 == g, hl, -jnp.inf), axis=0, keepdims=True)
            redr = jnp.max(jnp.where(bb1 == g, hr, -jnp.inf), axis=0, keepdims=True)
            red = jnp.maximum(redl, redr)
            pooled_ref[pl.ds(g, 1), :] = jnp.maximum(pooled_ref[pl.ds(g, 1), :], red)
            return carry

        lax.fori_loop(bmin, bmax + 1, seg_body, 0)

        @pl.when(i == _NBLK - 1)
        def _fin():
            pooled = pooled_ref[...]
            pooled = jnp.where(jnp.isfinite(pooled), pooled, 0.0)
            hh = jnp.maximum(_dot_t(pooled, wf1_ref[...]) + bf1_ref[...], 0.0)
            out_ref[...] = _dot_t(hh, wf2_ref[...]) + bf2_ref[...]

    full = lambda shape: pl.BlockSpec(shape, lambda i: tuple(0 for _ in shape))
    return pl.pallas_call(
        body,
        grid=(_NBLK,),
        in_specs=[
            full((1, 1024)), full((1, 1024)), full((1, 1024)), full((1, 1024)),
            full((256, 512)), full((1, 256)), full((10, 256)), full((1, 10)),
            pl.BlockSpec((_BLK, 1024), lambda i: (i, 0)),
            pl.BlockSpec((_BLK, 2), lambda i: (i, 0)),
        ],
        out_specs=full((_G, 10)),
        out_shape=jax.ShapeDtypeStruct((_G, 10), jnp.float32),
        scratch_shapes=[pltpu.VMEM((_G, 512), jnp.float32)],
    )(mu, rv, g3_t, be3_t, wf1, bf1, wf2, bf2, pre, bb)


def kernel(x, edge_index, batch, W1l, b1, W1r, g1, be1, W2l, b2, W2r, g2, be2,
           W3l, b3, W3r, g3, be3, Wf1, bf1, Wf2, bf2):
    f32 = jnp.float32
    src = edge_index[0]
    dst = edge_index[1]
    npad = _EPAD - _E
    ar = jnp.arange(npad, dtype=jnp.int32)
    pad_src = (ar * 37) % _N
    pad_dst = _N + ar % (_NPAD - _N)
    srcr = jnp.concatenate([src, pad_src]).reshape(_ROWS, _CH)
    dstr = jnp.concatenate([dst, pad_dst]).reshape(_ROWS, _CH)
    z16 = jnp.zeros((_NPAD, 16), f32)
    z64 = jnp.zeros((_NPAD, 64), f32)
    ones128 = jnp.ones((_CH, 16), f32)

    e8 = jnp.eye(8, dtype=f32)
    e2 = jnp.eye(2, dtype=f32)
    w1l_pk = jnp.kron(e8, W1l.T)
    w1r_pk = jnp.kron(e8, W1r.T)
    w2l_pk = jnp.kron(e8, W2l.T)
    w2r_pk = jnp.kron(e8, W2r.T)
    w3l_pk = jnp.kron(e2, W3l.T)
    w3r_pk = jnp.kron(e2, W3r.T)
    tf16 = jnp.kron(jnp.ones((8, 8), f32), jnp.eye(16, dtype=f32))
    tf64 = jnp.kron(jnp.ones((8, 8), f32), jnp.eye(64, dtype=f32))
    t8 = lambda v: jnp.tile(v, 8).reshape(1, -1)
    t2 = lambda v: jnp.tile(v, 2).reshape(1, -1)
    x2 = x.reshape(_NP8, 1024)

    y1p, z1p = _tk1(x2, w1l_pk, w1r_pk)
    p1 = _sc_agg(16, True)(y1p.reshape(_N, 16), srcr, dstr, z16, ones128)
    h1p, dinvp = _tk2(p1.reshape(2, 2 * _PP8, 128), z1p,
                      t8(b1), t8(g1), t8(be1), tf16)
    p2 = _sc_agg(16, False)(h1p.reshape(_N, 16), srcr, dstr, z16)
    h2p = _tk3(p2.reshape(2, _PP8, 128), h1p, w2l_pk, t8(b2), w2r_pk,
               t8(g2), t8(be2), dinvp, tf64)
    h2lin = h2p.reshape(_N, 64)
    p3 = _sc_agg(64, False)(h2lin, srcr, dstr, z64)
    root3 = _tk4r(h2lin.reshape(_NP2, 128), w3r_pk, t2(b3))
    dd = dinvp.reshape(_NP2, 32)
    pre3, mu3, rv3 = _tk4a(p3.reshape(2, _PP2, 128), root3, w3l_pk, dd)
    bb = batch.reshape(_NP2, 2)
    return _tk4b(pre3, mu3, rv3, t2(g3), t2(be3), bb,
                 Wf1, bf1.reshape(1, 256), Wf2, bf2.reshape(1, 10))


# speedup vs baseline: 1.0478x; 1.0478x over previous
import functools

import jax
import jax.numpy as jnp
from jax import lax
from jax.experimental import pallas as pl
from jax.experimental.pallas import tpu as pltpu
from jax.experimental.pallas import tpu_sc as plsc

_N = 10000
_E = 320000
_G = 64
_NPAD = 10240
_NW = 32
_CH = 128
_RPW = 80
_ROWS = _NW * _RPW
_EPAD = _ROWS * _CH
_NBUF = 8


def _sc_agg(d, with_deg):
    mesh = plsc.VectorSubcoreMesh(core_axis_name="c", subcore_axis_name="s")
    nacc = (2 * _NPAD) if with_deg else _NPAD
    rpt = _NPAD // 16

    scratch = [
        pltpu.VMEM((_RPW, _CH), jnp.int32),
        pltpu.VMEM((_RPW, _CH), jnp.int32),
        pltpu.VMEM((_NBUF, _CH, d), jnp.float32),
        pltpu.VMEM_SHARED((_NPAD, d), jnp.float32),
    ]
    scratch += [pltpu.SemaphoreType.DMA] * (2 * _NBUF)
    if with_deg:
        scratch += [pltpu.VMEM((_CH, d), jnp.float32),
                    pltpu.VMEM_SHARED((_NPAD, d), jnp.float32)]
        scratch += [pltpu.SemaphoreType.DMA] * _NBUF

    def body(*refs):
        if with_deg:
            (y_hbm, srcr_hbm, dstr_hbm, zeros_hbm, ones_hbm, out_hbm,
             sidx, didx, rows, acc, *sems) = refs
            gsems = sems[:_NBUF]
            ssems = sems[_NBUF:2 * _NBUF]
            ones, dacc, *s2sems = sems[2 * _NBUF:]
        else:
            (y_hbm, srcr_hbm, dstr_hbm, zeros_hbm, out_hbm,
             sidx, didx, rows, acc, *sems) = refs
            gsems = sems[:_NBUF]
            ssems = sems[_NBUF:2 * _NBUF]

        c = lax.axis_index("c")
        s = lax.axis_index("s")
        wid = s * 2 + c
        pltpu.sync_copy(zeros_hbm.at[pl.ds(s * rpt, rpt)], acc.at[pl.ds(s * rpt, rpt)])
        if with_deg:
            pltpu.sync_copy(zeros_hbm.at[pl.ds(s * rpt, rpt)], dacc.at[pl.ds(s * rpt, rpt)])
        base = wid * _RPW
        pltpu.sync_copy(srcr_hbm.at[pl.ds(base, _RPW)], sidx)
        pltpu.sync_copy(dstr_hbm.at[pl.ds(base, _RPW)], didx)
        if with_deg:
            pltpu.sync_copy(ones_hbm, ones)
        plsc.subcore_barrier()

        for b in range(_NBUF):
            pltpu.async_copy(y_hbm.at[sidx.at[b]], rows.at[b], gsems[b])

        def step(i, carry):
            for b in range(_NBUF):
                r = i * _NBUF + b
                pltpu.make_async_copy(y_hbm.at[sidx.at[r]], rows.at[b], gsems[b]).wait()
                sd = pltpu.async_copy(rows.at[b], acc.at[didx.at[r]], ssems[b], add=True)
                if with_deg:
                    sd2 = pltpu.async_copy(ones, dacc.at[didx.at[r]], s2sems[b], add=True)
                nxt = r + _NBUF

                @pl.when(nxt < _RPW)
                def _refill():
                    sd.wait()
                    if with_deg:
                        sd2.wait()
                    pltpu.async_copy(y_hbm.at[sidx.at[nxt]], rows.at[b], gsems[b])

            return carry

        lax.fori_loop(0, _RPW // _NBUF, step, 0)
        for b in range(_NBUF):
            r = _RPW - _NBUF + b
            pltpu.make_async_copy(rows.at[b], acc.at[didx.at[r]], ssems[b]).wait()
            if with_deg:
                pltpu.make_async_copy(ones, dacc.at[didx.at[r]], s2sems[b]).wait()
        plsc.subcore_barrier()
        pltpu.sync_copy(acc.at[pl.ds(s * rpt, rpt)],
                        out_hbm.at[c, pl.ds(s * rpt, rpt)])
        if with_deg:
            pltpu.sync_copy(dacc.at[pl.ds(s * rpt, rpt)],
                            out_hbm.at[c, pl.ds(_NPAD + s * rpt, rpt)])

    return functools.partial(
        pl.kernel,
        out_type=jax.ShapeDtypeStruct((2, nacc, d), jnp.float32),
        mesh=mesh,
        scratch_types=scratch,
        compiler_params=pltpu.CompilerParams(use_tc_tiling_on_sc=False),
    )(body)


def _mm(a, b):
    return lax.dot_general(a, b, (((1,), (0,)), ((), ())),
                           preferred_element_type=jnp.float32)


def _dot_t(a, b):
    return lax.dot_general(a, b, (((1,), (1,)), ((), ())),
                           preferred_element_type=jnp.float32)


_NP8 = _N // 8
_PP8 = _NPAD // 8
_NP2 = _N // 2
_PP2 = _NPAD // 2


def _tk1(x2, w1l_pk, w1r_pk):
    def body(x_ref, wl_ref, wr_ref, y_ref, z_ref):
        xv = x_ref[...]
        y_ref[...] = _mm(xv, wl_ref[...])
        z_ref[...] = _mm(xv, wr_ref[...])

    return pl.pallas_call(
        body,
        out_shape=(jax.ShapeDtypeStruct((_NP8, 128), jnp.float32),
                   jax.ShapeDtypeStruct((_NP8, 128), jnp.float32)),
    )(x2, w1l_pk, w1r_pk)


def _fold_bn(pre, tfold, n_nodes, g_t, be_t):
    s = jnp.sum(pre, axis=0, keepdims=True)
    sq = jnp.sum(pre * pre, axis=0, keepdims=True)
    mu = _mm(s, tfold) * (1.0 / n_nodes)
    ex2 = _mm(sq, tfold) * (1.0 / n_nodes)
    var = ex2 - mu * mu
    h = (pre - mu) * lax.rsqrt(var + 1e-5) * g_t + be_t
    return jnp.maximum(h, 0.0)


def _tk2(p, z1, b1_t, g1_t, be1_t, tf16):
    def body(p_ref, z_ref, b_ref, g_ref, be_ref, tf_ref, h_ref, dinv_ref):
        sm = p_ref[0] + p_ref[1]
        agg = sm[:_NP8, :]
        deg = sm[_PP8:_PP8 + _NP8, :]
        dinv = 1.0 / jnp.maximum(deg, 1.0)
        pre = agg * dinv + b_ref[...] + z_ref[...]
        h_ref[...] = _fold_bn(pre, tf_ref[...], _N, g_ref[...], be_ref[...])
        dinv_ref[...] = dinv

    return pl.pallas_call(
        body,
        out_shape=(jax.ShapeDtypeStruct((_NP8, 128), jnp.float32),
                   jax.ShapeDtypeStruct((_NP8, 128), jnp.float32)),
    )(p, z1, b1_t, g1_t, be1_t, tf16)


def _tk3(p, h1, w2l_pk, b2_t, w2r_pk, g2_t, be2_t, dinv, tf64):
    def body(p_ref, h1_ref, wl_ref, b_ref, wr_ref, g_ref, be_ref, dinv_ref,
             tf_ref, h2_ref):
        agg = (p_ref[0] + p_ref[1])[:_NP8, :]
        mean2 = agg * dinv_ref[...]
        pre = _mm(mean2, wl_ref[...]) + b_ref[...] + _mm(h1_ref[...], wr_ref[...])
        h2_ref[...] = _fold_bn(pre, tf_ref[...], _N, g_ref[...], be_ref[...])

    return pl.pallas_call(
        body,
        out_shape=jax.ShapeDtypeStruct((_NP8, 512), jnp.float32),
    )(p, h1, w2l_pk, b2_t, w2r_pk, g2_t, be2_t, dinv, tf64)


def _tk4a(p, h2pair, w3l_pk, w3r_pk, b3_t, dd):
    def body(p_ref, h2_ref, wl_ref, wr_ref, b_ref, dd_ref, pre_ref, mu_ref, rv_ref):
        agg = (p_ref[0] + p_ref[1])[:_NP2, :]
        mm = _mm(agg, wl_ref[...])
        dd = dd_ref[...]
        mean3 = jnp.concatenate(
            [mm[:, :512] * dd[:, 0:1], mm[:, 512:] * dd[:, 16:17]], axis=1)
        pre = mean3 + b_ref[...] + _mm(h2_ref[...], wr_ref[...])
        pre_ref[...] = pre
        s = jnp.sum(pre, axis=0, keepdims=True)
        sq = jnp.sum(pre * pre, axis=0, keepdims=True)
        sf = s[:, :512] + s[:, 512:]
        sqf = sq[:, :512] + sq[:, 512:]
        mu = jnp.concatenate([sf, sf], axis=1) * (1.0 / _N)
        ex2 = jnp.concatenate([sqf, sqf], axis=1) * (1.0 / _N)
        mu_ref[...] = mu
        rv_ref[...] = lax.rsqrt(ex2 - mu * mu + 1e-5)

    return pl.pallas_call(
        body,
        out_shape=(jax.ShapeDtypeStruct((_NP2, 1024), jnp.float32),
                   jax.ShapeDtypeStruct((1, 1024), jnp.float32),
                   jax.ShapeDtypeStruct((1, 1024), jnp.float32)),
    )(p, h2pair, w3l_pk, w3r_pk, b3_t, dd)


_BLK = 200
_NBLK = _NP2 // _BLK


def _tk4b(pre, mu, rv, g3_t, be3_t, bb, wf1, bf1, wf2, bf2):
    def body(mu_ref, rv_ref, g_ref, be_ref, wf1_ref, bf1_ref, wf2_ref, bf2_ref,
             pre_ref, bb_ref, out_ref, pooled_ref):
        i = pl.program_id(0)

        @pl.when(i == 0)
        def _init():
            pooled_ref[...] = jnp.full((_G, 512), -jnp.inf, jnp.float32)

        h = pre_ref[...]
        h = (h - mu_ref[...]) * rv_ref[...] * g_ref[...] + be_ref[...]
        h = jnp.maximum(h, 0.0)
        hl = h[:, :512]
        hr = h[:, 512:]
        bbv = bb_ref[...]
        bb0 = bbv[:, 0:1]
        bb1 = bbv[:, 1:2]
        bmin = jnp.min(bb0)
        bmax = jnp.max(bb1)

        def seg_body(g, carry):
            redl = jnp.max(jnp.where(bb0 == g, hl, -jnp.inf), axis=0, keepdims=True)
            redr = jnp.max(jnp.where(bb1 == g, hr, -jnp.inf), axis=0, keepdims=True)
            red = jnp.maximum(redl, redr)
            pooled_ref[pl.ds(g, 1), :] = jnp.maximum(pooled_ref[pl.ds(g, 1), :], red)
            return carry

        lax.fori_loop(bmin, bmax + 1, seg_body, 0)

        @pl.when(i == _NBLK - 1)
        def _fin():
            pooled = pooled_ref[...]
            pooled = jnp.where(jnp.isfinite(pooled), pooled, 0.0)
            hh = jnp.maximum(_dot_t(pooled, wf1_ref[...]) + bf1_ref[...], 0.0)
            out_ref[...] = _dot_t(hh, wf2_ref[...]) + bf2_ref[...]

    full = lambda shape: pl.BlockSpec(shape, lambda i: tuple(0 for _ in shape))
    return pl.pallas_call(
        body,
        grid=(_NBLK,),
        in_specs=[
            full((1, 1024)), full((1, 1024)), full((1, 1024)), full((1, 1024)),
            full((256, 512)), full((1, 256)), full((10, 256)), full((1, 10)),
            pl.BlockSpec((_BLK, 1024), lambda i: (i, 0)),
            pl.BlockSpec((_BLK, 2), lambda i: (i, 0)),
        ],
        out_specs=full((_G, 10)),
        out_shape=jax.ShapeDtypeStruct((_G, 10), jnp.float32),
        scratch_shapes=[pltpu.VMEM((_G, 512), jnp.float32)],
    )(mu, rv, g3_t, be3_t, wf1, bf1, wf2, bf2, pre, bb)


def kernel(x, edge_index, batch, W1l, b1, W1r, g1, be1, W2l, b2, W2r, g2, be2,
           W3l, b3, W3r, g3, be3, Wf1, bf1, Wf2, bf2):
    f32 = jnp.float32
    src = edge_index[0]
    dst = edge_index[1]
    npad = _EPAD - _E
    ar = jnp.arange(npad, dtype=jnp.int32)
    pad_src = (ar * 37) % _N
    pad_dst = _N + ar % (_NPAD - _N)
    srcr = jnp.concatenate([src, pad_src]).reshape(_ROWS, _CH)
    dstr = jnp.concatenate([dst, pad_dst]).reshape(_ROWS, _CH)
    z16 = jnp.zeros((_NPAD, 16), f32)
    z64 = jnp.zeros((_NPAD, 64), f32)
    ones128 = jnp.ones((_CH, 16), f32)

    e8 = jnp.eye(8, dtype=f32)
    e2 = jnp.eye(2, dtype=f32)
    w1l_pk = jnp.kron(e8, W1l.T)
    w1r_pk = jnp.kron(e8, W1r.T)
    w2l_pk = jnp.kron(e8, W2l.T)
    w2r_pk = jnp.kron(e8, W2r.T)
    w3l_pk = jnp.kron(e2, W3l.T)
    w3r_pk = jnp.kron(e2, W3r.T)
    tf16 = jnp.kron(jnp.ones((8, 8), f32), jnp.eye(16, dtype=f32))
    tf64 = jnp.kron(jnp.ones((8, 8), f32), jnp.eye(64, dtype=f32))
    t8 = lambda v: jnp.tile(v, 8).reshape(1, -1)
    t2 = lambda v: jnp.tile(v, 2).reshape(1, -1)
    x2 = x.reshape(_NP8, 1024)

    y1p, z1p = _tk1(x2, w1l_pk, w1r_pk)
    p1 = _sc_agg(16, True)(y1p.reshape(_N, 16), srcr, dstr, z16, ones128)
    h1p, dinvp = _tk2(p1.reshape(2, 2 * _PP8, 128), z1p,
                      t8(b1), t8(g1), t8(be1), tf16)
    p2 = _sc_agg(16, False)(h1p.reshape(_N, 16), srcr, dstr, z16)
    h2p = _tk3(p2.reshape(2, _PP8, 128), h1p, w2l_pk, t8(b2), w2r_pk,
               t8(g2), t8(be2), dinvp, tf64)
    h2lin = h2p.reshape(_N, 64)
    p3 = _sc_agg(64, False)(h2lin, srcr, dstr, z64)
    dd = dinvp.reshape(_NP2, 32)
    pre3, mu3, rv3 = _tk4a(p3.reshape(2, _PP2, 128), h2lin.reshape(_NP2, 128),
                           w3l_pk, w3r_pk, t2(b3), dd)
    bb = batch.reshape(_NP2, 2)
    return _tk4b(pre3, mu3, rv3, t2(g3), t2(be3), bb,
                 Wf1, bf1.reshape(1, 256), Wf2, bf2.reshape(1, 10))


# speedup vs baseline: 1.0819x; 1.0325x over previous
import functools

import jax
import jax.numpy as jnp
from jax import lax
from jax.experimental import pallas as pl
from jax.experimental.pallas import tpu as pltpu
from jax.experimental.pallas import tpu_sc as plsc

_N = 10000
_E = 320000
_G = 64
_NPAD = 10240
_NW = 32
_CH = 128
_ER = _E // _CH
_RW = _ER // _NW
_NBUF = 6


def _sc_agg(d, with_deg):
    mesh = plsc.VectorSubcoreMesh(core_axis_name="c", subcore_axis_name="s")
    nacc = (2 * _NPAD) if with_deg else _NPAD
    rpt = _NPAD // 16

    scratch = [
        pltpu.VMEM((_RW + 1, _CH), jnp.int32),
        pltpu.VMEM((_RW + 1, _CH), jnp.int32),
        pltpu.VMEM((_NBUF, _CH, d), jnp.float32),
        pltpu.VMEM_SHARED((_NPAD, d), jnp.float32),
    ]
    scratch += [pltpu.SemaphoreType.DMA] * (2 * _NBUF)
    if with_deg:
        scratch += [pltpu.VMEM((_CH, d), jnp.float32),
                    pltpu.VMEM_SHARED((_NPAD, d), jnp.float32)]
        scratch += [pltpu.SemaphoreType.DMA] * _NBUF

    def body(*refs):
        if with_deg:
            (y_hbm, er_hbm, zeros_hbm, ones_hbm, out_hbm,
             sidx, didx, rows, acc, *sems) = refs
            gsems = sems[:_NBUF]
            ssems = sems[_NBUF:2 * _NBUF]
            ones, dacc, *s2sems = sems[2 * _NBUF:]
        else:
            (y_hbm, er_hbm, zeros_hbm, out_hbm,
             sidx, didx, rows, acc, *sems) = refs
            gsems = sems[:_NBUF]
            ssems = sems[_NBUF:2 * _NBUF]

        c = lax.axis_index("c")
        s = lax.axis_index("s")
        wid = s * 2 + c
        has_tail = wid < (_ER - _NW * _RW)
        pltpu.sync_copy(zeros_hbm.at[pl.ds(s * rpt, rpt)], acc.at[pl.ds(s * rpt, rpt)])
        if with_deg:
            pltpu.sync_copy(zeros_hbm.at[pl.ds(s * rpt, rpt)], dacc.at[pl.ds(s * rpt, rpt)])
        base = wid * _RW
        pltpu.sync_copy(er_hbm.at[0, pl.ds(base, _RW)], sidx.at[pl.ds(0, _RW)])
        pltpu.sync_copy(er_hbm.at[1, pl.ds(base, _RW)], didx.at[pl.ds(0, _RW)])

        @pl.when(has_tail)
        def _tail_pre():
            tr = _NW * _RW + wid
            pltpu.sync_copy(er_hbm.at[0, pl.ds(tr, 1)], sidx.at[pl.ds(_RW, 1)])
            pltpu.sync_copy(er_hbm.at[1, pl.ds(tr, 1)], didx.at[pl.ds(_RW, 1)])

        if with_deg:
            pltpu.sync_copy(ones_hbm, ones)
        plsc.subcore_barrier()

        for b in range(_NBUF):
            pltpu.async_copy(y_hbm.at[sidx.at[b]], rows.at[b], gsems[b])

        def step(i, carry):
            for b in range(_NBUF):
                r = i * _NBUF + b
                pltpu.make_async_copy(y_hbm.at[sidx.at[r]], rows.at[b], gsems[b]).wait()
                sd = pltpu.async_copy(rows.at[b], acc.at[didx.at[r]], ssems[b], add=True)
                if with_deg:
                    sd2 = pltpu.async_copy(ones, dacc.at[didx.at[r]], s2sems[b], add=True)
                nxt = r + _NBUF

                @pl.when(nxt < _RW)
                def _refill():
                    sd.wait()
                    if with_deg:
                        sd2.wait()
                    pltpu.async_copy(y_hbm.at[sidx.at[nxt]], rows.at[b], gsems[b])

            return carry

        lax.fori_loop(0, _RW // _NBUF, step, 0)
        for b in range(_NBUF):
            r = _RW - _NBUF + b
            pltpu.make_async_copy(rows.at[b], acc.at[didx.at[r]], ssems[b]).wait()
            if with_deg:
                pltpu.make_async_copy(ones, dacc.at[didx.at[r]], s2sems[b]).wait()

        @pl.when(has_tail)
        def _tail_run():
            pltpu.async_copy(y_hbm.at[sidx.at[_RW]], rows.at[0], gsems[0]).wait()
            pltpu.async_copy(rows.at[0], acc.at[didx.at[_RW]], ssems[0], add=True).wait()
            if with_deg:
                pltpu.async_copy(ones, dacc.at[didx.at[_RW]], s2sems[0], add=True).wait()

        plsc.subcore_barrier()
        pltpu.sync_copy(acc.at[pl.ds(s * rpt, rpt)],
                        out_hbm.at[c, pl.ds(s * rpt, rpt)])
        if with_deg:
            pltpu.sync_copy(dacc.at[pl.ds(s * rpt, rpt)],
                            out_hbm.at[c, pl.ds(_NPAD + s * rpt, rpt)])

    return functools.partial(
        pl.kernel,
        out_type=jax.ShapeDtypeStruct((2, nacc, d), jnp.float32),
        mesh=mesh,
        scratch_types=scratch,
        compiler_params=pltpu.CompilerParams(use_tc_tiling_on_sc=False),
    )(body)


def _mm(a, b):
    return lax.dot_general(a, b, (((1,), (0,)), ((), ())),
                           preferred_element_type=jnp.float32)


def _dot_t(a, b):
    return lax.dot_general(a, b, (((1,), (1,)), ((), ())),
                           preferred_element_type=jnp.float32)


_NP8 = _N // 8
_PP8 = _NPAD // 8
_NP2 = _N // 2
_PP2 = _NPAD // 2


def _tk1(x2, w1l_pk, w1r_pk):
    def body(x_ref, wl_ref, wr_ref, y_ref, z_ref):
        xv = x_ref[...]
        y_ref[...] = _mm(xv, wl_ref[...])
        z_ref[...] = _mm(xv, wr_ref[...])

    return pl.pallas_call(
        body,
        out_shape=(jax.ShapeDtypeStruct((_NP8, 128), jnp.float32),
                   jax.ShapeDtypeStruct((_NP8, 128), jnp.float32)),
    )(x2, w1l_pk, w1r_pk)


def _fold_bn(pre, tfold, n_nodes, g_t, be_t):
    s = jnp.sum(pre, axis=0, keepdims=True)
    sq = jnp.sum(pre * pre, axis=0, keepdims=True)
    mu = _mm(s, tfold) * (1.0 / n_nodes)
    ex2 = _mm(sq, tfold) * (1.0 / n_nodes)
    var = ex2 - mu * mu
    h = (pre - mu) * lax.rsqrt(var + 1e-5) * g_t + be_t
    return jnp.maximum(h, 0.0)


def _tk2(p, z1, b1_t, g1_t, be1_t, tf16):
    def body(p_ref, z_ref, b_ref, g_ref, be_ref, tf_ref, h_ref, dinv_ref):
        sm = p_ref[0] + p_ref[1]
        agg = sm[:_NP8, :]
        deg = sm[_PP8:_PP8 + _NP8, :]
        dinv = 1.0 / jnp.maximum(deg, 1.0)
        pre = agg * dinv + b_ref[...] + z_ref[...]
        h_ref[...] = _fold_bn(pre, tf_ref[...], _N, g_ref[...], be_ref[...])
        dinv_ref[...] = dinv

    return pl.pallas_call(
        body,
        out_shape=(jax.ShapeDtypeStruct((_NP8, 128), jnp.float32),
                   jax.ShapeDtypeStruct((_NP8, 128), jnp.float32)),
    )(p, z1, b1_t, g1_t, be1_t, tf16)


def _tk3(p, h1, w2l_pk, b2_t, w2r_pk, g2_t, be2_t, dinv, tf64):
    def body(p_ref, h1_ref, wl_ref, b_ref, wr_ref, g_ref, be_ref, dinv_ref,
             tf_ref, h2_ref):
        agg = (p_ref[0] + p_ref[1])[:_NP8, :]
        mean2 = agg * dinv_ref[...]
        pre = _mm(mean2, wl_ref[...]) + b_ref[...] + _mm(h1_ref[...], wr_ref[...])
        h2_ref[...] = _fold_bn(pre, tf_ref[...], _N, g_ref[...], be_ref[...])

    return pl.pallas_call(
        body,
        out_shape=jax.ShapeDtypeStruct((_NP8, 512), jnp.float32),
    )(p, h1, w2l_pk, b2_t, w2r_pk, g2_t, be2_t, dinv, tf64)


def _tk4a(p, h2pair, w3l_pk, w3r_pk, b3_t, dd):
    def body(p_ref, h2_ref, wl_ref, wr_ref, b_ref, dd_ref, pre_ref, mu_ref, rv_ref):
        agg = (p_ref[0] + p_ref[1])[:_NP2, :]
        mm = _mm(agg, wl_ref[...])
        dd = dd_ref[...]
        mean3 = jnp.concatenate(
            [mm[:, :512] * dd[:, 0:1], mm[:, 512:] * dd[:, 16:17]], axis=1)
        pre = mean3 + b_ref[...] + _mm(h2_ref[...], wr_ref[...])
        pre_ref[...] = pre
        s = jnp.sum(pre, axis=0, keepdims=True)
        sq = jnp.sum(pre * pre, axis=0, keepdims=True)
        sf = s[:, :512] + s[:, 512:]
        sqf = sq[:, :512] + sq[:, 512:]
        mu = jnp.concatenate([sf, sf], axis=1) * (1.0 / _N)
        ex2 = jnp.concatenate([sqf, sqf], axis=1) * (1.0 / _N)
        mu_ref[...] = mu
        rv_ref[...] = lax.rsqrt(ex2 - mu * mu + 1e-5)

    return pl.pallas_call(
        body,
        out_shape=(jax.ShapeDtypeStruct((_NP2, 1024), jnp.float32),
                   jax.ShapeDtypeStruct((1, 1024), jnp.float32),
                   jax.ShapeDtypeStruct((1, 1024), jnp.float32)),
    )(p, h2pair, w3l_pk, w3r_pk, b3_t, dd)


_BLK = 200
_NBLK = _NP2 // _BLK


def _tk4b(pre, mu, rv, g3_t, be3_t, bb, wf1, bf1, wf2, bf2):
    def body(mu_ref, rv_ref, g_ref, be_ref, wf1_ref, bf1_ref, wf2_ref, bf2_ref,
             pre_ref, bb_ref, out_ref, pooled_ref):
        i = pl.program_id(0)

        @pl.when(i == 0)
        def _init():
            pooled_ref[...] = jnp.full((_G, 512), -jnp.inf, jnp.float32)

        h = pre_ref[...]
        h = (h - mu_ref[...]) * rv_ref[...] * g_ref[...] + be_ref[...]
        h = jnp.maximum(h, 0.0)
        hl = h[:, :512]
        hr = h[:, 512:]
        bbv = bb_ref[...]
        bb0 = bbv[:, 0:1]
        bb1 = bbv[:, 1:2]
        bmin = jnp.min(bb0)
        bmax = jnp.max(bb1)

        def seg_body(g, carry):
            redl = jnp.max(jnp.where(bb0 == g, hl, -jnp.inf), axis=0, keepdims=True)
            redr = jnp.max(jnp.where(bb1 == g, hr, -jnp.inf), axis=0, keepdims=True)
            red = jnp.maximum(redl, redr)
            pooled_ref[pl.ds(g, 1), :] = jnp.maximum(pooled_ref[pl.ds(g, 1), :], red)
            return carry

        lax.fori_loop(bmin, bmax + 1, seg_body, 0)

        @pl.when(i == _NBLK - 1)
        def _fin():
            pooled = pooled_ref[...]
            pooled = jnp.where(jnp.isfinite(pooled), pooled, 0.0)
            hh = jnp.maximum(_dot_t(pooled, wf1_ref[...]) + bf1_ref[...], 0.0)
            out_ref[...] = _dot_t(hh, wf2_ref[...]) + bf2_ref[...]

    full = lambda shape: pl.BlockSpec(shape, lambda i: tuple(0 for _ in shape))
    return pl.pallas_call(
        body,
        grid=(_NBLK,),
        in_specs=[
            full((1, 1024)), full((1, 1024)), full((1, 1024)), full((1, 1024)),
            full((256, 512)), full((1, 256)), full((10, 256)), full((1, 10)),
            pl.BlockSpec((_BLK, 1024), lambda i: (i, 0)),
            pl.BlockSpec((_BLK, 2), lambda i: (i, 0)),
        ],
        out_specs=full((_G, 10)),
        out_shape=jax.ShapeDtypeStruct((_G, 10), jnp.float32),
        scratch_shapes=[pltpu.VMEM((_G, 512), jnp.float32)],
    )(mu, rv, g3_t, be3_t, wf1, bf1, wf2, bf2, pre, bb)


def kernel(x, edge_index, batch, W1l, b1, W1r, g1, be1, W2l, b2, W2r, g2, be2,
           W3l, b3, W3r, g3, be3, Wf1, bf1, Wf2, bf2):
    f32 = jnp.float32
    er = edge_index.reshape(2, _ER, _CH)
    z16 = jnp.zeros((_NPAD, 16), f32)
    z64 = jnp.zeros((_NPAD, 64), f32)
    ones128 = jnp.ones((_CH, 16), f32)

    e8 = jnp.eye(8, dtype=f32)
    e2 = jnp.eye(2, dtype=f32)
    w1l_pk = jnp.kron(e8, W1l.T)
    w1r_pk = jnp.kron(e8, W1r.T)
    w2l_pk = jnp.kron(e8, W2l.T)
    w2r_pk = jnp.kron(e8, W2r.T)
    w3l_pk = jnp.kron(e2, W3l.T)
    w3r_pk = jnp.kron(e2, W3r.T)
    tf16 = jnp.kron(jnp.ones((8, 8), f32), jnp.eye(16, dtype=f32))
    tf64 = jnp.kron(jnp.ones((8, 8), f32), jnp.eye(64, dtype=f32))
    t8 = lambda v: jnp.tile(v, 8).reshape(1, -1)
    t2 = lambda v: jnp.tile(v, 2).reshape(1, -1)
    x2 = x.reshape(_NP8, 1024)

    y1p, z1p = _tk1(x2, w1l_pk, w1r_pk)
    p1 = _sc_agg(16, True)(y1p.reshape(_N, 16), er, z16, ones128)
    h1p, dinvp = _tk2(p1.reshape(2, 2 * _PP8, 128), z1p,
                      t8(b1), t8(g1), t8(be1), tf16)
    p2 = _sc_agg(16, False)(h1p.reshape(_N, 16), er, z16)
    h2p = _tk3(p2.reshape(2, _PP8, 128), h1p, w2l_pk, t8(b2), w2r_pk,
               t8(g2), t8(be2), dinvp, tf64)
    h2lin = h2p.reshape(_N, 64)
    p3 = _sc_agg(64, False)(h2lin, er, z64)
    dd = dinvp.reshape(_NP2, 32)
    pre3, mu3, rv3 = _tk4a(p3.reshape(2, _PP2, 128), h2lin.reshape(_NP2, 128),
                           w3l_pk, w3r_pk, t2(b3), dd)
    bb = batch.reshape(_NP2, 2)
    return _tk4b(pre3, mu3, rv3, t2(g3), t2(be3), bb,
                 Wf1, bf1.reshape(1, 256), Wf2, bf2.reshape(1, 10))
